# baseline (speedup 1.0000x reference)
import functools

import jax
import jax.numpy as jnp
from jax import lax
from jax.experimental import pallas as pl
from jax.experimental.pallas import tpu as pltpu
from jax.experimental.pallas import tpu_sc as plsc

B = 1024
FEAT = 3 * 64 * 64


@functools.lru_cache(maxsize=1)
def _make_sc_gather():
    info = plsc.get_sparse_core_info()
    nc, ns = info.num_cores, info.num_subcores
    nw = nc * ns
    bpw = B // nw

    mesh = plsc.VectorSubcoreMesh(core_axis_name="c", subcore_axis_name="s")

    @functools.partial(
        pl.kernel,
        mesh=mesh,
        out_type=[
            jax.ShapeDtypeStruct((B,), jnp.float32),
            jax.ShapeDtypeStruct((B,), jnp.float32),
        ],
        scratch_types=[
            pltpu.VMEM((bpw,), jnp.int32),
            pltpu.VMEM((bpw,), jnp.float32),
            pltpu.VMEM((bpw,), jnp.float32),
            pltpu.SemaphoreType.DMA,
        ],
    )
    def sc_gather(t_hbm, sa_hbm, sb_hbm, out_a_hbm, out_b_hbm, idx_v, a_v, b_v, sem):
        wid = lax.axis_index("s") * nc + lax.axis_index("c")
        base = wid * bpw
        pltpu.sync_copy(t_hbm.at[pl.ds(base, bpw)], idx_v)
        ca = pltpu.async_copy(sa_hbm.at[idx_v], a_v, sem)
        cb = pltpu.async_copy(sb_hbm.at[idx_v], b_v, sem)
        ca.wait()
        cb.wait()
        pltpu.sync_copy(a_v, out_a_hbm.at[pl.ds(base, bpw)])
        pltpu.sync_copy(b_v, out_b_hbm.at[pl.ds(base, bpw)])

    return sc_gather


def _tc_body(sa_ref, sb_ref, x0_ref, eps_ref, out_ref):
    out_ref[...] = sa_ref[...] * x0_ref[...] + sb_ref[...] * eps_ref[...]


def _tc_apply(sa_g, sb_g, x0, eps):
    bb = 32
    c, h, w = x0.shape[1:]
    grid = (B // bb,)
    coef_spec = pl.BlockSpec((bb, 1, 1, 1), lambda i: (i, 0, 0, 0))
    img_spec = pl.BlockSpec((bb, c, h, w), lambda i: (i, 0, 0, 0))
    return pl.pallas_call(
        _tc_body,
        grid=grid,
        in_specs=[coef_spec, coef_spec, img_spec, img_spec],
        out_specs=img_spec,
        out_shape=jax.ShapeDtypeStruct(x0.shape, jnp.float32),
    )(sa_g.reshape(B, 1, 1, 1), sb_g.reshape(B, 1, 1, 1), x0, eps)


def kernel(x0, t, eps, sqrt_alpha_bar, sqrt_one_minus_alpha_bar):
    sa_g, sb_g = _make_sc_gather()(t.astype(jnp.int32), sqrt_alpha_bar,
                            sqrt_one_minus_alpha_bar)
    return _tc_apply(sa_g, sb_g, x0, eps)


# speedup vs baseline: 5.1545x; 5.1545x over previous
import functools

import jax
import jax.numpy as jnp
from jax import lax
from jax.experimental import pallas as pl
from jax.experimental.pallas import tpu as pltpu
from jax.experimental.pallas import tpu_sc as plsc

B = 1024
FEAT = 3 * 64 * 64


@functools.lru_cache(maxsize=1)
def _make_sc_gather():
    info = plsc.get_sparse_core_info()
    nc, ns = info.num_cores, info.num_subcores
    nw = nc * ns
    bpw = B // nw

    mesh = plsc.VectorSubcoreMesh(core_axis_name="c", subcore_axis_name="s")

    @functools.partial(
        pl.kernel,
        mesh=mesh,
        out_type=[
            jax.ShapeDtypeStruct((B,), jnp.float32),
            jax.ShapeDtypeStruct((B,), jnp.float32),
        ],
        scratch_types=[
            pltpu.VMEM((bpw,), jnp.int32),
            pltpu.VMEM((bpw,), jnp.float32),
            pltpu.VMEM((bpw,), jnp.float32),
            pltpu.SemaphoreType.DMA,
        ],
    )
    def sc_gather(t_hbm, sa_hbm, sb_hbm, out_a_hbm, out_b_hbm, idx_v, a_v, b_v, sem):
        wid = lax.axis_index("s") * nc + lax.axis_index("c")
        base = wid * bpw
        pltpu.sync_copy(t_hbm.at[pl.ds(base, bpw)], idx_v)
        ca = pltpu.async_copy(sa_hbm.at[idx_v], a_v, sem)
        cb = pltpu.async_copy(sb_hbm.at[idx_v], b_v, sem)
        ca.wait()
        cb.wait()
        pltpu.sync_copy(a_v, out_a_hbm.at[pl.ds(base, bpw)])
        pltpu.sync_copy(b_v, out_b_hbm.at[pl.ds(base, bpw)])

    return sc_gather


def _tc_body(sa_ref, sb_ref, x0_ref, eps_ref, out_ref):
    out_ref[...] = sa_ref[...] * x0_ref[...] + sb_ref[...] * eps_ref[...]


def _tc_apply(sa_g, sb_g, x0, eps):
    c, h, w = x0.shape[1:]
    xt = x0.transpose(1, 2, 3, 0).reshape(FEAT, B)
    et = eps.transpose(1, 2, 3, 0).reshape(FEAT, B)
    rb = 512
    grid = (FEAT // rb,)
    coef_spec = pl.BlockSpec((1, B), lambda i: (0, 0))
    img_spec = pl.BlockSpec((rb, B), lambda i: (i, 0))
    out = pl.pallas_call(
        _tc_body,
        grid=grid,
        in_specs=[coef_spec, coef_spec, img_spec, img_spec],
        out_specs=img_spec,
        out_shape=jax.ShapeDtypeStruct((FEAT, B), jnp.float32),
    )(sa_g.reshape(1, B), sb_g.reshape(1, B), xt, et)
    return out.reshape(c, h, w, B).transpose(3, 0, 1, 2)


def kernel(x0, t, eps, sqrt_alpha_bar, sqrt_one_minus_alpha_bar):
    sa_g, sb_g = _make_sc_gather()(t.astype(jnp.int32), sqrt_alpha_bar,
                            sqrt_one_minus_alpha_bar)
    return _tc_apply(sa_g, sb_g, x0, eps)


# speedup vs baseline: 5.2448x; 1.0175x over previous
import functools

import jax
import jax.numpy as jnp
from jax import lax
from jax.experimental import pallas as pl
from jax.experimental.pallas import tpu as pltpu
from jax.experimental.pallas import tpu_sc as plsc

B = 1024
FEAT = 3 * 64 * 64


@functools.lru_cache(maxsize=1)
def _make_sc_gather():
    info = plsc.get_sparse_core_info()
    nc, ns = info.num_cores, info.num_subcores
    nw = nc * ns
    bpw = B // nw

    mesh = plsc.VectorSubcoreMesh(core_axis_name="c", subcore_axis_name="s")

    @functools.partial(
        pl.kernel,
        mesh=mesh,
        out_type=[
            jax.ShapeDtypeStruct((B,), jnp.float32),
            jax.ShapeDtypeStruct((B,), jnp.float32),
        ],
        scratch_types=[
            pltpu.VMEM((bpw,), jnp.int32),
            pltpu.VMEM((bpw,), jnp.float32),
            pltpu.VMEM((bpw,), jnp.float32),
            pltpu.SemaphoreType.DMA,
        ],
    )
    def sc_gather(t_hbm, sa_hbm, sb_hbm, out_a_hbm, out_b_hbm, idx_v, a_v, b_v, sem):
        wid = lax.axis_index("s") * nc + lax.axis_index("c")
        base = wid * bpw
        pltpu.sync_copy(t_hbm.at[pl.ds(base, bpw)], idx_v)
        ca = pltpu.async_copy(sa_hbm.at[idx_v], a_v, sem)
        cb = pltpu.async_copy(sb_hbm.at[idx_v], b_v, sem)
        ca.wait()
        cb.wait()
        pltpu.sync_copy(a_v, out_a_hbm.at[pl.ds(base, bpw)])
        pltpu.sync_copy(b_v, out_b_hbm.at[pl.ds(base, bpw)])

    return sc_gather


def _tc_body(sa_ref, sb_ref, x0_ref, eps_ref, out_ref):
    out_ref[...] = sa_ref[...] * x0_ref[...] + sb_ref[...] * eps_ref[...]


def _tc_apply(sa_g, sb_g, x0, eps):
    c, h, w = x0.shape[1:]
    xt = x0.transpose(1, 2, 3, 0).reshape(FEAT, B)
    et = eps.transpose(1, 2, 3, 0).reshape(FEAT, B)
    rb = 1024
    grid = (FEAT // rb,)
    coef_spec = pl.BlockSpec((1, B), lambda i: (0, 0))
    img_spec = pl.BlockSpec((rb, B), lambda i: (i, 0))
    out = pl.pallas_call(
        _tc_body,
        grid=grid,
        in_specs=[coef_spec, coef_spec, img_spec, img_spec],
        out_specs=img_spec,
        out_shape=jax.ShapeDtypeStruct((FEAT, B), jnp.float32),
    )(sa_g.reshape(1, B), sb_g.reshape(1, B), xt, et)
    return out.reshape(c, h, w, B).transpose(3, 0, 1, 2)


def kernel(x0, t, eps, sqrt_alpha_bar, sqrt_one_minus_alpha_bar):
    sa_g, sb_g = _make_sc_gather()(t.astype(jnp.int32), sqrt_alpha_bar,
                            sqrt_one_minus_alpha_bar)
    return _tc_apply(sa_g, sb_g, x0, eps)


# speedup vs baseline: 5.2653x; 1.0039x over previous
import functools

import jax
import jax.numpy as jnp
from jax import lax
from jax.experimental import pallas as pl
from jax.experimental.pallas import tpu as pltpu
from jax.experimental.pallas import tpu_sc as plsc

B = 1024
FEAT = 3 * 64 * 64


@functools.lru_cache(maxsize=1)
def _make_sc_gather():
    info = plsc.get_sparse_core_info()
    nc, ns = info.num_cores, info.num_subcores
    nw = nc * ns
    bpw = B // nw

    mesh = plsc.VectorSubcoreMesh(core_axis_name="c", subcore_axis_name="s")

    @functools.partial(
        pl.kernel,
        mesh=mesh,
        out_type=[
            jax.ShapeDtypeStruct((B,), jnp.float32),
            jax.ShapeDtypeStruct((B,), jnp.float32),
        ],
        scratch_types=[
            pltpu.VMEM((bpw,), jnp.int32),
            pltpu.VMEM((bpw,), jnp.float32),
            pltpu.VMEM((bpw,), jnp.float32),
            pltpu.SemaphoreType.DMA,
        ],
    )
    def sc_gather(t_hbm, sa_hbm, sb_hbm, out_a_hbm, out_b_hbm, idx_v, a_v, b_v, sem):
        wid = lax.axis_index("s") * nc + lax.axis_index("c")
        base = wid * bpw
        pltpu.sync_copy(t_hbm.at[pl.ds(base, bpw)], idx_v)
        ca = pltpu.async_copy(sa_hbm.at[idx_v], a_v, sem)
        cb = pltpu.async_copy(sb_hbm.at[idx_v], b_v, sem)
        ca.wait()
        cb.wait()
        pltpu.sync_copy(a_v, out_a_hbm.at[pl.ds(base, bpw)])
        pltpu.sync_copy(b_v, out_b_hbm.at[pl.ds(base, bpw)])

    return sc_gather


def _tc_body(sa_ref, sb_ref, x0_ref, eps_ref, out_ref):
    out_ref[...] = sa_ref[...] * x0_ref[...] + sb_ref[...] * eps_ref[...]


def _tc_apply(sa_g, sb_g, x0, eps):
    c, h, w = x0.shape[1:]
    xt = x0.transpose(1, 2, 3, 0).reshape(FEAT, B)
    et = eps.transpose(1, 2, 3, 0).reshape(FEAT, B)
    rb = 2048
    grid = (FEAT // rb,)
    coef_spec = pl.BlockSpec((1, B), lambda i: (0, 0))
    img_spec = pl.BlockSpec((rb, B), lambda i: (i, 0))
    out = pl.pallas_call(
        _tc_body,
        grid=grid,
        in_specs=[coef_spec, coef_spec, img_spec, img_spec],
        out_specs=img_spec,
        out_shape=jax.ShapeDtypeStruct((FEAT, B), jnp.float32),
    )(sa_g.reshape(1, B), sb_g.reshape(1, B), xt, et)
    return out.reshape(c, h, w, B).transpose(3, 0, 1, 2)


def kernel(x0, t, eps, sqrt_alpha_bar, sqrt_one_minus_alpha_bar):
    sa_g, sb_g = _make_sc_gather()(t.astype(jnp.int32), sqrt_alpha_bar,
                            sqrt_one_minus_alpha_bar)
    return _tc_apply(sa_g, sb_g, x0, eps)
